# Initial kernel scaffold; baseline (speedup 1.0000x reference)
import jax
import jax.numpy as jnp
from jax.experimental import pallas as pl


def kernel(h, edge_index, W1, b1, W2, b2):
    raise NotImplementedError("write your pallas kernel here")


# speedup vs baseline: 15.2453x; 15.2453x over previous
import functools

import jax
import jax.numpy as jnp
from jax import lax
from jax.experimental import pallas as pl
from jax.experimental.pallas import tpu as pltpu
from jax.experimental.pallas import tpu_sc as plsc

_CHUNK = 128


def _sc_degrees(n_nodes, kpt, nc, ns):
    np1 = n_nodes + 1
    mesh = plsc.VectorSubcoreMesh(core_axis_name="c", subcore_axis_name="s")

    @functools.partial(
        pl.kernel,
        mesh=mesh,
        compiler_params=pltpu.CompilerParams(use_tc_tiling_on_sc=False),
        out_type=jax.ShapeDtypeStruct((2, 2, np1), jnp.float32),
        scratch_types=[
            pltpu.VMEM((kpt, _CHUNK), jnp.int32),
            pltpu.VMEM((kpt, _CHUNK), jnp.int32),
            pltpu.VMEM((_CHUNK,), jnp.float32),
            pltpu.VMEM_SHARED((np1,), jnp.float32),
            pltpu.VMEM_SHARED((np1,), jnp.float32),
        ],
    )
    def deg_kernel(src_hbm, dst_hbm, zero_hbm, out_hbm, sbuf, dbuf, ones_v,
                   oacc, iacc):
        c = lax.axis_index("c")
        s = lax.axis_index("s")
        g = s * nc + c
        for k in range(_CHUNK // 16):
            ones_v[pl.ds(16 * k, 16)] = jnp.ones((16,), jnp.float32)

        @pl.when(s == 0)
        def _init():
            pltpu.sync_copy(zero_hbm, oacc)
            pltpu.sync_copy(zero_hbm, iacc)

        plsc.subcore_barrier()
        pltpu.sync_copy(src_hbm.at[g], sbuf)
        pltpu.sync_copy(dst_hbm.at[g], dbuf)

        def step(j, carry):
            pltpu.sync_copy(ones_v, oacc.at[sbuf.at[j]], add=True)
            pltpu.sync_copy(ones_v, iacc.at[dbuf.at[j]], add=True)
            return carry

        lax.fori_loop(0, kpt, step, 0)
        plsc.subcore_barrier()

        @pl.when(s == 0)
        def _writeout():
            pltpu.sync_copy(oacc, out_hbm.at[c, 0])
            pltpu.sync_copy(iacc, out_hbm.at[c, 1])

    return deg_kernel


def _sc_edge_rows(n_nodes, width, kpt, nc, ns):
    np1 = n_nodes + 1
    mesh = plsc.VectorSubcoreMesh(core_axis_name="c", subcore_axis_name="s")

    @functools.partial(
        pl.kernel,
        mesh=mesh,
        compiler_params=pltpu.CompilerParams(use_tc_tiling_on_sc=False),
        out_type=jax.ShapeDtypeStruct((2, np1, width), jnp.float32),
        scratch_types=[
            pltpu.VMEM((kpt, _CHUNK), jnp.int32),
            pltpu.VMEM((kpt, _CHUNK), jnp.int32),
            pltpu.VMEM((_CHUNK, width), jnp.float32),
            pltpu.VMEM_SHARED((np1, width), jnp.float32),
            pltpu.SemaphoreType.DMA,
        ],
    )
    def rows_kernel(src_hbm, dst_hbm, z_hbm, zero_hbm, out_hbm, sbuf, dbuf,
                    rows_v, acc, sem):
        c = lax.axis_index("c")
        s = lax.axis_index("s")
        g = s * nc + c

        @pl.when(s == 0)
        def _init():
            pltpu.sync_copy(zero_hbm, acc)

        plsc.subcore_barrier()
        pltpu.sync_copy(src_hbm.at[g], sbuf)
        pltpu.sync_copy(dst_hbm.at[g], dbuf)

        def step(j, carry):
            pltpu.async_copy(z_hbm.at[sbuf.at[j]], rows_v, sem).wait()
            pltpu.sync_copy(rows_v, acc.at[dbuf.at[j]], add=True)
            return carry

        lax.fori_loop(0, kpt, step, 0)
        plsc.subcore_barrier()

        @pl.when(s == 0)
        def _writeout():
            pltpu.sync_copy(acc, out_hbm.at[c])

    return rows_kernel


def _sc_edge_scalar(n_nodes, kpt, nc, ns):
    np1 = n_nodes + 1
    mesh = plsc.VectorSubcoreMesh(core_axis_name="c", subcore_axis_name="s")

    @functools.partial(
        pl.kernel,
        mesh=mesh,
        compiler_params=pltpu.CompilerParams(use_tc_tiling_on_sc=False),
        out_type=jax.ShapeDtypeStruct((2, np1), jnp.float32),
        scratch_types=[
            pltpu.VMEM((kpt, _CHUNK), jnp.int32),
            pltpu.VMEM((kpt, _CHUNK), jnp.int32),
            pltpu.VMEM((_CHUNK,), jnp.float32),
            pltpu.VMEM_SHARED((np1,), jnp.float32),
            pltpu.SemaphoreType.DMA,
        ],
    )
    def scal_kernel(src_hbm, dst_hbm, v_hbm, zero_hbm, out_hbm, sbuf, dbuf,
                    vals_v, acc, sem):
        c = lax.axis_index("c")
        s = lax.axis_index("s")
        g = s * nc + c

        @pl.when(s == 0)
        def _init():
            pltpu.sync_copy(zero_hbm, acc)

        plsc.subcore_barrier()
        pltpu.sync_copy(src_hbm.at[g], sbuf)
        pltpu.sync_copy(dst_hbm.at[g], dbuf)

        def step(j, carry):
            pltpu.async_copy(v_hbm.at[sbuf.at[j]], vals_v, sem).wait()
            pltpu.sync_copy(vals_v, acc.at[dbuf.at[j]], add=True)
            return carry

        lax.fori_loop(0, kpt, step, 0)
        plsc.subcore_barrier()

        @pl.when(s == 0)
        def _writeout():
            pltpu.sync_copy(acc, out_hbm.at[c])

    return scal_kernel


def _tc_matmul_body(h_ref, w_ref, o_ref):
    o_ref[...] = jnp.dot(h_ref[...], w_ref[...],
                         preferred_element_type=jnp.float32)


def _tc_mid1_body(deg_ref, z0_ref, z_ref, on_ref, in_ref):
    out_deg = deg_ref[0, 0] + deg_ref[1, 0]
    in_deg = deg_ref[0, 1] + deg_ref[1, 1]
    onorm = lax.rsqrt(jnp.maximum(out_deg, 1.0))
    inorm = lax.rsqrt(jnp.maximum(in_deg, 1.0))
    on_ref[...] = onorm
    in_ref[...] = inorm
    z_ref[...] = z0_ref[...] * onorm[:, None]


def _tc_mid2_body(agg_ref, on_ref, in_ref, b1_ref, w2_ref, v_ref):
    agg = agg_ref[0] + agg_ref[1]
    pre = agg * in_ref[...][:, None] + b1_ref[...]
    h1 = jnp.where(pre > 0, pre, jnp.exp(jnp.minimum(pre, 0.0)) - 1.0)
    w2m = jnp.mean(w2_ref[...], axis=1)
    v_ref[...] = jnp.sum(h1 * w2m[None, :], axis=1) * on_ref[...]


def _tc_final_body(s_ref, in_ref, b2_ref, o_ref):
    o_ref[...] = (s_ref[0] + s_ref[1]) * in_ref[...] + jnp.mean(b2_ref[...])


def kernel(h, edge_index, W1, b1, W2, b2):
    n, _ = h.shape
    e = edge_index.shape[1]
    hid = W1.shape[1]
    np1 = n + 1
    f32 = jnp.float32

    info = plsc.get_sparse_core_info()
    nc, ns = info.num_cores, info.num_subcores
    nw = nc * ns

    per_tile = -(-e // (nw * _CHUNK)) * _CHUNK
    ep = per_tile * nw
    kpt = per_tile // _CHUNK
    if ep != e:
        pad = jnp.full((2, ep - e), n, dtype=jnp.int32)
        epad = jnp.concatenate([edge_index.astype(jnp.int32), pad], axis=1)
    else:
        epad = edge_index.astype(jnp.int32)
    src3 = epad[0].reshape(nw, kpt, _CHUNK)
    dst3 = epad[1].reshape(nw, kpt, _CHUNK)

    zeros_1 = jnp.zeros((np1,), f32)
    zeros_w = jnp.zeros((np1, hid), f32)

    deg2 = _sc_degrees(n, kpt, nc, ns)(src3, dst3, zeros_1)
    z0 = pl.pallas_call(
        _tc_matmul_body,
        out_shape=jax.ShapeDtypeStruct((n, hid), f32),
    )(h, W1)

    z, onorm, inorm = pl.pallas_call(
        _tc_mid1_body,
        out_shape=(
            jax.ShapeDtypeStruct((n, hid), f32),
            jax.ShapeDtypeStruct((n,), f32),
            jax.ShapeDtypeStruct((n,), f32),
        ),
    )(deg2[:, :, :n], z0)

    z_p = jnp.concatenate([z, jnp.zeros((1, hid), f32)], axis=0)
    agg2 = _sc_edge_rows(n, hid, kpt, nc, ns)(src3, dst3, z_p, zeros_w)

    v = pl.pallas_call(
        _tc_mid2_body,
        out_shape=jax.ShapeDtypeStruct((n,), f32),
    )(agg2[:, :n, :], onorm, inorm, b1.reshape(1, hid), W2)

    v_p = jnp.concatenate([v, jnp.zeros((1,), f32)])
    s2 = _sc_edge_scalar(n, kpt, nc, ns)(src3, dst3, v_p, zeros_1)

    logits = pl.pallas_call(
        _tc_final_body,
        out_shape=jax.ShapeDtypeStruct((n,), f32),
    )(s2[:, :n], inorm, b2.reshape(1, -1))
    return logits


# speedup vs baseline: 29.6980x; 1.9480x over previous
import functools

import jax
import jax.numpy as jnp
from jax import lax
from jax.experimental import pallas as pl
from jax.experimental.pallas import tpu as pltpu
from jax.experimental.pallas import tpu_sc as plsc

_CHUNK = 128


def _sc_degrees(n_nodes, kpt, nc, ns):
    np1 = n_nodes + 1
    mesh = plsc.VectorSubcoreMesh(core_axis_name="c", subcore_axis_name="s")

    @functools.partial(
        pl.kernel,
        mesh=mesh,
        compiler_params=pltpu.CompilerParams(use_tc_tiling_on_sc=False),
        out_type=jax.ShapeDtypeStruct((2, 2, np1), jnp.float32),
        scratch_types=[
            pltpu.VMEM((kpt, _CHUNK), jnp.int32),
            pltpu.VMEM((kpt, _CHUNK), jnp.int32),
            pltpu.VMEM((_CHUNK,), jnp.float32),
            pltpu.VMEM_SHARED((np1,), jnp.float32),
            pltpu.VMEM_SHARED((np1,), jnp.float32),
        ],
    )
    def deg_kernel(src_hbm, dst_hbm, zero_hbm, out_hbm, sbuf, dbuf, ones_v,
                   oacc, iacc):
        c = lax.axis_index("c")
        s = lax.axis_index("s")
        g = s * nc + c
        for k in range(_CHUNK // 16):
            ones_v[pl.ds(16 * k, 16)] = jnp.ones((16,), jnp.float32)

        @pl.when(s == 0)
        def _init():
            pltpu.sync_copy(zero_hbm, oacc)
            pltpu.sync_copy(zero_hbm, iacc)

        plsc.subcore_barrier()
        pltpu.sync_copy(src_hbm.at[g], sbuf)
        pltpu.sync_copy(dst_hbm.at[g], dbuf)

        def step(j, carry):
            pltpu.sync_copy(ones_v, oacc.at[sbuf.at[j]], add=True)
            pltpu.sync_copy(ones_v, iacc.at[dbuf.at[j]], add=True)
            return carry

        lax.fori_loop(0, kpt, step, 0)
        plsc.subcore_barrier()

        @pl.when(s == 0)
        def _writeout():
            pltpu.sync_copy(oacc, out_hbm.at[c, 0])
            pltpu.sync_copy(iacc, out_hbm.at[c, 1])

    return deg_kernel


def _sc_edge_rows(n_nodes, width, kpt, nc, ns):
    np1 = n_nodes + 1
    mesh = plsc.VectorSubcoreMesh(core_axis_name="c", subcore_axis_name="s")

    @functools.partial(
        pl.kernel,
        mesh=mesh,
        compiler_params=pltpu.CompilerParams(use_tc_tiling_on_sc=False),
        out_type=jax.ShapeDtypeStruct((2, np1, width), jnp.float32),
        scratch_types=[
            pltpu.VMEM((kpt, _CHUNK), jnp.int32),
            pltpu.VMEM((kpt, _CHUNK), jnp.int32),
            pltpu.VMEM((2, _CHUNK, width), jnp.float32),
            pltpu.VMEM_SHARED((np1, width), jnp.float32),
            pltpu.VMEM_SHARED((np1, width), jnp.float32),
            pltpu.SemaphoreType.DMA((2,)),
            pltpu.SemaphoreType.DMA((2,)),
        ],
    )
    def rows_kernel(src_hbm, dst_hbm, z_hbm, zero_hbm, out_hbm, sbuf, dbuf,
                    rows2, z_sp, acc, gsems, ssems):
        c = lax.axis_index("c")
        s = lax.axis_index("s")
        g = s * nc + c

        @pl.when(s == 0)
        def _init():
            pltpu.sync_copy(zero_hbm, acc)

        @pl.when(s == 1)
        def _stage():
            pltpu.sync_copy(z_hbm, z_sp)

        plsc.subcore_barrier()
        pltpu.sync_copy(src_hbm.at[g], sbuf)
        pltpu.sync_copy(dst_hbm.at[g], dbuf)

        pltpu.async_copy(z_sp.at[sbuf.at[0]], rows2.at[0], gsems.at[0])

        def step(j, carry):
            slot = j % 2
            nslot = (j + 1) % 2

            @pl.when(j >= 1)
            def _drain_prev_scatter():
                pltpu.make_async_copy(z_hbm.at[pl.ds(0, _CHUNK)],
                                      rows2.at[nslot],
                                      ssems.at[nslot]).wait()

            @pl.when(j + 1 < kpt)
            def _issue_next_gather():
                pltpu.async_copy(z_sp.at[sbuf.at[j + 1]], rows2.at[nslot],
                                 gsems.at[nslot])

            pltpu.make_async_copy(z_hbm.at[pl.ds(0, _CHUNK)], rows2.at[slot],
                                  gsems.at[slot]).wait()
            pltpu.async_copy(rows2.at[slot], acc.at[dbuf.at[j]],
                             ssems.at[slot], add=True)
            return carry

        lax.fori_loop(0, kpt, step, 0)
        pltpu.make_async_copy(z_hbm.at[pl.ds(0, _CHUNK)], rows2.at[(kpt - 1) % 2],
                              ssems.at[(kpt - 1) % 2]).wait()
        plsc.subcore_barrier()

        @pl.when(s == 0)
        def _writeout():
            pltpu.sync_copy(acc, out_hbm.at[c])

    return rows_kernel


def _sc_edge_scalar(n_nodes, kpt, nc, ns):
    np1 = n_nodes + 1
    mesh = plsc.VectorSubcoreMesh(core_axis_name="c", subcore_axis_name="s")

    @functools.partial(
        pl.kernel,
        mesh=mesh,
        compiler_params=pltpu.CompilerParams(use_tc_tiling_on_sc=False),
        out_type=jax.ShapeDtypeStruct((2, np1), jnp.float32),
        scratch_types=[
            pltpu.VMEM((kpt, _CHUNK), jnp.int32),
            pltpu.VMEM((kpt, _CHUNK), jnp.int32),
            pltpu.VMEM((2, _CHUNK), jnp.float32),
            pltpu.VMEM_SHARED((np1,), jnp.float32),
            pltpu.VMEM_SHARED((np1,), jnp.float32),
            pltpu.SemaphoreType.DMA((2,)),
            pltpu.SemaphoreType.DMA((2,)),
        ],
    )
    def scal_kernel(src_hbm, dst_hbm, v_hbm, zero_hbm, out_hbm, sbuf, dbuf,
                    vals2, v_sp, acc, gsems, ssems):
        c = lax.axis_index("c")
        s = lax.axis_index("s")
        g = s * nc + c

        @pl.when(s == 0)
        def _init():
            pltpu.sync_copy(zero_hbm, acc)

        @pl.when(s == 1)
        def _stage():
            pltpu.sync_copy(v_hbm, v_sp)

        plsc.subcore_barrier()
        pltpu.sync_copy(src_hbm.at[g], sbuf)
        pltpu.sync_copy(dst_hbm.at[g], dbuf)

        pltpu.async_copy(v_sp.at[sbuf.at[0]], vals2.at[0], gsems.at[0])

        def step(j, carry):
            slot = j % 2
            nslot = (j + 1) % 2

            @pl.when(j >= 1)
            def _drain_prev_scatter():
                pltpu.make_async_copy(v_hbm.at[pl.ds(0, _CHUNK)],
                                      vals2.at[nslot],
                                      ssems.at[nslot]).wait()

            @pl.when(j + 1 < kpt)
            def _issue_next_gather():
                pltpu.async_copy(v_sp.at[sbuf.at[j + 1]], vals2.at[nslot],
                                 gsems.at[nslot])

            pltpu.make_async_copy(v_hbm.at[pl.ds(0, _CHUNK)], vals2.at[slot],
                                  gsems.at[slot]).wait()
            pltpu.async_copy(vals2.at[slot], acc.at[dbuf.at[j]],
                             ssems.at[slot], add=True)
            return carry

        lax.fori_loop(0, kpt, step, 0)
        pltpu.make_async_copy(v_hbm.at[pl.ds(0, _CHUNK)], vals2.at[(kpt - 1) % 2],
                              ssems.at[(kpt - 1) % 2]).wait()
        plsc.subcore_barrier()

        @pl.when(s == 0)
        def _writeout():
            pltpu.sync_copy(acc, out_hbm.at[c])

    return scal_kernel


def _tc_matmul_body(h_ref, w_ref, o_ref):
    o_ref[...] = jnp.dot(h_ref[...], w_ref[...],
                         preferred_element_type=jnp.float32)


def _tc_mid1_body(deg_ref, z0_ref, z_ref, on_ref, in_ref):
    out_deg = deg_ref[0, 0] + deg_ref[1, 0]
    in_deg = deg_ref[0, 1] + deg_ref[1, 1]
    onorm = lax.rsqrt(jnp.maximum(out_deg, 1.0))
    inorm = lax.rsqrt(jnp.maximum(in_deg, 1.0))
    on_ref[...] = onorm
    in_ref[...] = inorm
    z_ref[...] = z0_ref[...] * onorm[:, None]


def _tc_mid2_body(agg_ref, on_ref, in_ref, b1_ref, w2_ref, v_ref):
    agg = agg_ref[0] + agg_ref[1]
    pre = agg * in_ref[...][:, None] + b1_ref[...]
    h1 = jnp.where(pre > 0, pre, jnp.exp(jnp.minimum(pre, 0.0)) - 1.0)
    w2m = jnp.mean(w2_ref[...], axis=1)
    v_ref[...] = jnp.sum(h1 * w2m[None, :], axis=1) * on_ref[...]


def _tc_final_body(s_ref, in_ref, b2_ref, o_ref):
    o_ref[...] = (s_ref[0] + s_ref[1]) * in_ref[...] + jnp.mean(b2_ref[...])


def kernel(h, edge_index, W1, b1, W2, b2):
    n, _ = h.shape
    e = edge_index.shape[1]
    hid = W1.shape[1]
    np1 = n + 1
    f32 = jnp.float32

    info = plsc.get_sparse_core_info()
    nc, ns = info.num_cores, info.num_subcores
    nw = nc * ns

    per_tile = -(-e // (nw * _CHUNK)) * _CHUNK
    ep = per_tile * nw
    kpt = per_tile // _CHUNK
    if ep != e:
        pad = jnp.full((2, ep - e), n, dtype=jnp.int32)
        epad = jnp.concatenate([edge_index.astype(jnp.int32), pad], axis=1)
    else:
        epad = edge_index.astype(jnp.int32)
    src3 = epad[0].reshape(nw, kpt, _CHUNK)
    dst3 = epad[1].reshape(nw, kpt, _CHUNK)

    zeros_1 = jnp.zeros((np1,), f32)
    zeros_w = jnp.zeros((np1, hid), f32)

    deg2 = _sc_degrees(n, kpt, nc, ns)(src3, dst3, zeros_1)
    z0 = pl.pallas_call(
        _tc_matmul_body,
        out_shape=jax.ShapeDtypeStruct((n, hid), f32),
    )(h, W1)

    z, onorm, inorm = pl.pallas_call(
        _tc_mid1_body,
        out_shape=(
            jax.ShapeDtypeStruct((n, hid), f32),
            jax.ShapeDtypeStruct((n,), f32),
            jax.ShapeDtypeStruct((n,), f32),
        ),
    )(deg2[:, :, :n], z0)

    z_p = jnp.concatenate([z, jnp.zeros((1, hid), f32)], axis=0)
    agg2 = _sc_edge_rows(n, hid, kpt, nc, ns)(src3, dst3, z_p, zeros_w)

    v = pl.pallas_call(
        _tc_mid2_body,
        out_shape=jax.ShapeDtypeStruct((n,), f32),
    )(agg2[:, :n, :], onorm, inorm, b1.reshape(1, hid), W2)

    v_p = jnp.concatenate([v, jnp.zeros((1,), f32)])
    s2 = _sc_edge_scalar(n, kpt, nc, ns)(src3, dst3, v_p, zeros_1)

    logits = pl.pallas_call(
        _tc_final_body,
        out_shape=jax.ShapeDtypeStruct((n,), f32),
    )(s2[:, :n], inorm, b2.reshape(1, -1))
    return logits


# speedup vs baseline: 34.2960x; 1.1548x over previous
import functools

import jax
import jax.numpy as jnp
from jax import lax
from jax.experimental import pallas as pl
from jax.experimental.pallas import tpu as pltpu
from jax.experimental.pallas import tpu_sc as plsc

_CHUNK = 128


def _sc_degrees(n_nodes, kpt, nc, ns):
    np1 = n_nodes + 1
    mesh = plsc.VectorSubcoreMesh(core_axis_name="c", subcore_axis_name="s")

    @functools.partial(
        pl.kernel,
        mesh=mesh,
        compiler_params=pltpu.CompilerParams(use_tc_tiling_on_sc=False),
        out_type=jax.ShapeDtypeStruct((2, 2, np1), jnp.float32),
        scratch_types=[
            pltpu.VMEM((kpt, _CHUNK), jnp.int32),
            pltpu.VMEM((kpt, _CHUNK), jnp.int32),
            pltpu.VMEM((_CHUNK,), jnp.float32),
            pltpu.VMEM_SHARED((np1,), jnp.float32),
            pltpu.VMEM_SHARED((np1,), jnp.float32),
            pltpu.SemaphoreType.DMA,
            pltpu.SemaphoreType.DMA,
        ],
    )
    def deg_kernel(src_hbm, dst_hbm, zero_hbm, out_hbm, sbuf, dbuf, ones_v,
                   oacc, iacc, osem, isem):
        c = lax.axis_index("c")
        s = lax.axis_index("s")
        g = s * nc + c
        for k in range(_CHUNK // 16):
            ones_v[pl.ds(16 * k, 16)] = jnp.ones((16,), jnp.float32)

        @pl.when(s == 0)
        def _init():
            pltpu.sync_copy(zero_hbm, oacc)
            pltpu.sync_copy(zero_hbm, iacc)

        plsc.subcore_barrier()
        pltpu.sync_copy(src_hbm.at[g], sbuf)
        pltpu.sync_copy(dst_hbm.at[g], dbuf)

        lag = 4

        def step(j, carry):
            @pl.when(j >= lag)
            def _drain_old():
                pltpu.make_async_copy(zero_hbm.at[pl.ds(0, _CHUNK)], ones_v, osem).wait()
                pltpu.make_async_copy(zero_hbm.at[pl.ds(0, _CHUNK)], ones_v, isem).wait()

            pltpu.async_copy(ones_v, oacc.at[sbuf.at[j]], osem, add=True)
            pltpu.async_copy(ones_v, iacc.at[dbuf.at[j]], isem, add=True)
            return carry

        lax.fori_loop(0, kpt, step, 0)

        def drain(j, carry):
            pltpu.make_async_copy(zero_hbm.at[pl.ds(0, _CHUNK)], ones_v, osem).wait()
            pltpu.make_async_copy(zero_hbm.at[pl.ds(0, _CHUNK)], ones_v, isem).wait()
            return carry

        lax.fori_loop(0, min(lag, kpt), drain, 0)
        plsc.subcore_barrier()

        @pl.when(s == 0)
        def _writeout():
            pltpu.sync_copy(oacc, out_hbm.at[c, 0])
            pltpu.sync_copy(iacc, out_hbm.at[c, 1])

    return deg_kernel


def _sc_edge_rows(n_nodes, width, kpt, nc, ns):
    np1 = n_nodes + 1
    mesh = plsc.VectorSubcoreMesh(core_axis_name="c", subcore_axis_name="s")

    @functools.partial(
        pl.kernel,
        mesh=mesh,
        compiler_params=pltpu.CompilerParams(use_tc_tiling_on_sc=False),
        out_type=jax.ShapeDtypeStruct((2, np1, width), jnp.float32),
        scratch_types=[
            pltpu.VMEM((kpt, _CHUNK), jnp.int32),
            pltpu.VMEM((kpt, _CHUNK), jnp.int32),
            pltpu.VMEM((2, _CHUNK, width), jnp.float32),
            pltpu.VMEM_SHARED((np1, width), jnp.float32),
            pltpu.VMEM_SHARED((np1, width), jnp.float32),
            pltpu.SemaphoreType.DMA((2,)),
            pltpu.SemaphoreType.DMA((2,)),
        ],
    )
    def rows_kernel(src_hbm, dst_hbm, z_hbm, zero_hbm, out_hbm, sbuf, dbuf,
                    rows2, z_sp, acc, gsems, ssems):
        c = lax.axis_index("c")
        s = lax.axis_index("s")
        g = s * nc + c

        @pl.when(s == 0)
        def _init():
            pltpu.sync_copy(zero_hbm, acc)

        @pl.when(s == 1)
        def _stage():
            pltpu.sync_copy(z_hbm, z_sp)

        plsc.subcore_barrier()
        pltpu.sync_copy(src_hbm.at[g], sbuf)
        pltpu.sync_copy(dst_hbm.at[g], dbuf)

        pltpu.async_copy(z_sp.at[sbuf.at[0]], rows2.at[0], gsems.at[0])

        def step(j, carry):
            slot = j % 2
            nslot = (j + 1) % 2

            @pl.when(j >= 1)
            def _drain_prev_scatter():
                pltpu.make_async_copy(z_hbm.at[pl.ds(0, _CHUNK)],
                                      rows2.at[nslot],
                                      ssems.at[nslot]).wait()

            @pl.when(j + 1 < kpt)
            def _issue_next_gather():
                pltpu.async_copy(z_sp.at[sbuf.at[j + 1]], rows2.at[nslot],
                                 gsems.at[nslot])

            pltpu.make_async_copy(z_hbm.at[pl.ds(0, _CHUNK)], rows2.at[slot],
                                  gsems.at[slot]).wait()
            pltpu.async_copy(rows2.at[slot], acc.at[dbuf.at[j]],
                             ssems.at[slot], add=True)
            return carry

        lax.fori_loop(0, kpt, step, 0)
        pltpu.make_async_copy(z_hbm.at[pl.ds(0, _CHUNK)], rows2.at[(kpt - 1) % 2],
                              ssems.at[(kpt - 1) % 2]).wait()
        plsc.subcore_barrier()

        @pl.when(s == 0)
        def _writeout():
            pltpu.sync_copy(acc, out_hbm.at[c])

    return rows_kernel


def _sc_edge_scalar(n_nodes, kpt, nc, ns):
    np1 = n_nodes + 1
    mesh = plsc.VectorSubcoreMesh(core_axis_name="c", subcore_axis_name="s")

    @functools.partial(
        pl.kernel,
        mesh=mesh,
        compiler_params=pltpu.CompilerParams(use_tc_tiling_on_sc=False),
        out_type=jax.ShapeDtypeStruct((2, np1), jnp.float32),
        scratch_types=[
            pltpu.VMEM((kpt, _CHUNK), jnp.int32),
            pltpu.VMEM((kpt, _CHUNK), jnp.int32),
            pltpu.VMEM((2, _CHUNK), jnp.float32),
            pltpu.VMEM_SHARED((np1,), jnp.float32),
            pltpu.VMEM_SHARED((np1,), jnp.float32),
            pltpu.SemaphoreType.DMA((2,)),
            pltpu.SemaphoreType.DMA((2,)),
        ],
    )
    def scal_kernel(src_hbm, dst_hbm, v_hbm, zero_hbm, out_hbm, sbuf, dbuf,
                    vals2, v_sp, acc, gsems, ssems):
        c = lax.axis_index("c")
        s = lax.axis_index("s")
        g = s * nc + c

        @pl.when(s == 0)
        def _init():
            pltpu.sync_copy(zero_hbm, acc)

        @pl.when(s == 1)
        def _stage():
            pltpu.sync_copy(v_hbm, v_sp)

        plsc.subcore_barrier()
        pltpu.sync_copy(src_hbm.at[g], sbuf)
        pltpu.sync_copy(dst_hbm.at[g], dbuf)

        pltpu.async_copy(v_sp.at[sbuf.at[0]], vals2.at[0], gsems.at[0])

        def step(j, carry):
            slot = j % 2
            nslot = (j + 1) % 2

            @pl.when(j >= 1)
            def _drain_prev_scatter():
                pltpu.make_async_copy(v_hbm.at[pl.ds(0, _CHUNK)],
                                      vals2.at[nslot],
                                      ssems.at[nslot]).wait()

            @pl.when(j + 1 < kpt)
            def _issue_next_gather():
                pltpu.async_copy(v_sp.at[sbuf.at[j + 1]], vals2.at[nslot],
                                 gsems.at[nslot])

            pltpu.make_async_copy(v_hbm.at[pl.ds(0, _CHUNK)], vals2.at[slot],
                                  gsems.at[slot]).wait()
            pltpu.async_copy(vals2.at[slot], acc.at[dbuf.at[j]],
                             ssems.at[slot], add=True)
            return carry

        lax.fori_loop(0, kpt, step, 0)
        pltpu.make_async_copy(v_hbm.at[pl.ds(0, _CHUNK)], vals2.at[(kpt - 1) % 2],
                              ssems.at[(kpt - 1) % 2]).wait()
        plsc.subcore_barrier()

        @pl.when(s == 0)
        def _writeout():
            pltpu.sync_copy(acc, out_hbm.at[c])

    return scal_kernel


def _tc_a_body(n, deg_ref, h_ref, w1_ref, z_ref, on_ref, in_ref):
    out_deg = deg_ref[0, 0, pl.ds(0, n)] + deg_ref[1, 0, pl.ds(0, n)]
    in_deg = deg_ref[0, 1, pl.ds(0, n)] + deg_ref[1, 1, pl.ds(0, n)]
    onorm = lax.rsqrt(jnp.maximum(out_deg, 1.0))
    inorm = lax.rsqrt(jnp.maximum(in_deg, 1.0))
    on_ref[...] = onorm
    in_ref[...] = inorm
    z0 = jnp.dot(h_ref[...], w1_ref[...], preferred_element_type=jnp.float32)
    z_ref[pl.ds(0, n), :] = z0 * onorm[:, None]
    z_ref[pl.ds(n, 1), :] = jnp.zeros((1, z_ref.shape[1]), jnp.float32)


def _tc_b_body(n, agg_ref, on_ref, in_ref, b1_ref, w2_ref, v_ref):
    agg = agg_ref[0, pl.ds(0, n), :] + agg_ref[1, pl.ds(0, n), :]
    pre = agg * in_ref[...][:, None] + b1_ref[...]
    h1 = jnp.where(pre > 0, pre, jnp.exp(jnp.minimum(pre, 0.0)) - 1.0)
    w2m = jnp.mean(w2_ref[...], axis=1)
    v_ref[pl.ds(0, n)] = jnp.sum(h1 * w2m[None, :], axis=1) * on_ref[...]
    v_ref[pl.ds(n, 1)] = jnp.zeros((1,), jnp.float32)


def _tc_c_body(n, s_ref, in_ref, b2_ref, o_ref):
    o_ref[...] = ((s_ref[0, pl.ds(0, n)] + s_ref[1, pl.ds(0, n)])
                  * in_ref[...] + jnp.mean(b2_ref[...]))


def kernel(h, edge_index, W1, b1, W2, b2):
    n, _ = h.shape
    e = edge_index.shape[1]
    hid = W1.shape[1]
    np1 = n + 1
    f32 = jnp.float32

    info = plsc.get_sparse_core_info()
    nc, ns = info.num_cores, info.num_subcores
    nw = nc * ns

    per_tile = -(-e // (nw * _CHUNK)) * _CHUNK
    ep = per_tile * nw
    kpt = per_tile // _CHUNK
    if ep != e:
        pad = jnp.full((2, ep - e), n, dtype=jnp.int32)
        epad = jnp.concatenate([edge_index.astype(jnp.int32), pad], axis=1)
    else:
        epad = edge_index.astype(jnp.int32)
    src3 = epad[0].reshape(nw, kpt, _CHUNK)
    dst3 = epad[1].reshape(nw, kpt, _CHUNK)

    zeros_1 = jnp.zeros((np1,), f32)
    zeros_w = jnp.zeros((np1, hid), f32)

    deg2 = _sc_degrees(n, kpt, nc, ns)(src3, dst3, zeros_1)

    z_p, onorm, inorm = pl.pallas_call(
        functools.partial(_tc_a_body, n),
        out_shape=(
            jax.ShapeDtypeStruct((np1, hid), f32),
            jax.ShapeDtypeStruct((n,), f32),
            jax.ShapeDtypeStruct((n,), f32),
        ),
    )(deg2, h, W1)

    agg2 = _sc_edge_rows(n, hid, kpt, nc, ns)(src3, dst3, z_p, zeros_w)

    v_p = pl.pallas_call(
        functools.partial(_tc_b_body, n),
        out_shape=jax.ShapeDtypeStruct((np1,), f32),
    )(agg2, onorm, inorm, b1.reshape(1, hid), W2)

    s2 = _sc_edge_scalar(n, kpt, nc, ns)(src3, dst3, v_p, zeros_1)

    logits = pl.pallas_call(
        functools.partial(_tc_c_body, n),
        out_shape=jax.ShapeDtypeStruct((n,), f32),
    )(s2, inorm, b2.reshape(1, -1))
    return logits


# speedup vs baseline: 34.3463x; 1.0015x over previous
import functools

import jax
import jax.numpy as jnp
from jax import lax
from jax.experimental import pallas as pl
from jax.experimental.pallas import tpu as pltpu
from jax.experimental.pallas import tpu_sc as plsc

_CHUNK = 128


def _sc_degrees(n_nodes, kpt, nc, ns):
    np1 = n_nodes + 1
    mesh = plsc.VectorSubcoreMesh(core_axis_name="c", subcore_axis_name="s")

    @functools.partial(
        pl.kernel,
        mesh=mesh,
        compiler_params=pltpu.CompilerParams(use_tc_tiling_on_sc=False, needs_layout_passes=False),
        out_type=jax.ShapeDtypeStruct((2, 2, np1), jnp.float32),
        scratch_types=[
            pltpu.VMEM((kpt, _CHUNK), jnp.int32),
            pltpu.VMEM((kpt, _CHUNK), jnp.int32),
            pltpu.VMEM((_CHUNK,), jnp.float32),
            pltpu.VMEM_SHARED((np1,), jnp.float32),
            pltpu.VMEM_SHARED((np1,), jnp.float32),
            pltpu.SemaphoreType.DMA,
            pltpu.SemaphoreType.DMA,
        ],
    )
    def deg_kernel(src_hbm, dst_hbm, zero_hbm, out_hbm, sbuf, dbuf, ones_v,
                   oacc, iacc, osem, isem):
        c = lax.axis_index("c")
        s = lax.axis_index("s")
        g = s * nc + c
        for k in range(_CHUNK // 16):
            ones_v[pl.ds(16 * k, 16)] = jnp.ones((16,), jnp.float32)

        @pl.when(s == 0)
        def _init():
            pltpu.sync_copy(zero_hbm, oacc)
            pltpu.sync_copy(zero_hbm, iacc)

        plsc.subcore_barrier()
        pltpu.sync_copy(src_hbm.at[g], sbuf)
        pltpu.sync_copy(dst_hbm.at[g], dbuf)

        lag = 4

        def step(j, carry):
            @pl.when(j >= lag)
            def _drain_old():
                pltpu.make_async_copy(zero_hbm.at[pl.ds(0, _CHUNK)], ones_v, osem).wait()
                pltpu.make_async_copy(zero_hbm.at[pl.ds(0, _CHUNK)], ones_v, isem).wait()

            pltpu.async_copy(ones_v, oacc.at[sbuf.at[j]], osem, add=True)
            pltpu.async_copy(ones_v, iacc.at[dbuf.at[j]], isem, add=True)
            return carry

        lax.fori_loop(0, kpt, step, 0)

        def drain(j, carry):
            pltpu.make_async_copy(zero_hbm.at[pl.ds(0, _CHUNK)], ones_v, osem).wait()
            pltpu.make_async_copy(zero_hbm.at[pl.ds(0, _CHUNK)], ones_v, isem).wait()
            return carry

        lax.fori_loop(0, min(lag, kpt), drain, 0)
        plsc.subcore_barrier()

        @pl.when(s == 0)
        def _writeout():
            pltpu.sync_copy(oacc, out_hbm.at[c, 0])
            pltpu.sync_copy(iacc, out_hbm.at[c, 1])

    return deg_kernel


def _sc_edge_rows(n_nodes, width, kpt, nc, ns):
    np1 = n_nodes + 1
    mesh = plsc.VectorSubcoreMesh(core_axis_name="c", subcore_axis_name="s")

    @functools.partial(
        pl.kernel,
        mesh=mesh,
        compiler_params=pltpu.CompilerParams(use_tc_tiling_on_sc=False, needs_layout_passes=False),
        out_type=jax.ShapeDtypeStruct((2, np1, width), jnp.float32),
        scratch_types=[
            pltpu.VMEM((kpt, _CHUNK), jnp.int32),
            pltpu.VMEM((kpt, _CHUNK), jnp.int32),
            pltpu.VMEM((2, _CHUNK, width), jnp.float32),
            pltpu.VMEM_SHARED((np1, width), jnp.float32),
            pltpu.VMEM_SHARED((np1, width), jnp.float32),
            pltpu.SemaphoreType.DMA((2,)),
            pltpu.SemaphoreType.DMA((2,)),
        ],
    )
    def rows_kernel(src_hbm, dst_hbm, z_hbm, zero_hbm, out_hbm, sbuf, dbuf,
                    rows2, z_sp, acc, gsems, ssems):
        c = lax.axis_index("c")
        s = lax.axis_index("s")
        g = s * nc + c

        @pl.when(s == 0)
        def _init():
            pltpu.sync_copy(zero_hbm, acc)

        @pl.when(s == 1)
        def _stage():
            pltpu.sync_copy(z_hbm, z_sp)

        plsc.subcore_barrier()
        pltpu.sync_copy(src_hbm.at[g], sbuf)
        pltpu.sync_copy(dst_hbm.at[g], dbuf)

        pltpu.async_copy(z_sp.at[sbuf.at[0]], rows2.at[0], gsems.at[0])

        def step(j, carry):
            slot = j % 2
            nslot = (j + 1) % 2

            @pl.when(j >= 1)
            def _drain_prev_scatter():
                pltpu.make_async_copy(z_hbm.at[pl.ds(0, _CHUNK)],
                                      rows2.at[nslot],
                                      ssems.at[nslot]).wait()

            @pl.when(j + 1 < kpt)
            def _issue_next_gather():
                pltpu.async_copy(z_sp.at[sbuf.at[j + 1]], rows2.at[nslot],
                                 gsems.at[nslot])

            pltpu.make_async_copy(z_hbm.at[pl.ds(0, _CHUNK)], rows2.at[slot],
                                  gsems.at[slot]).wait()
            pltpu.async_copy(rows2.at[slot], acc.at[dbuf.at[j]],
                             ssems.at[slot], add=True)
            return carry

        lax.fori_loop(0, kpt, step, 0)
        pltpu.make_async_copy(z_hbm.at[pl.ds(0, _CHUNK)], rows2.at[(kpt - 1) % 2],
                              ssems.at[(kpt - 1) % 2]).wait()
        plsc.subcore_barrier()

        @pl.when(s == 0)
        def _writeout():
            pltpu.sync_copy(acc, out_hbm.at[c])

    return rows_kernel


def _sc_edge_scalar(n_nodes, kpt, nc, ns):
    np1 = n_nodes + 1
    mesh = plsc.VectorSubcoreMesh(core_axis_name="c", subcore_axis_name="s")

    @functools.partial(
        pl.kernel,
        mesh=mesh,
        compiler_params=pltpu.CompilerParams(use_tc_tiling_on_sc=False, needs_layout_passes=False),
        out_type=jax.ShapeDtypeStruct((2, np1), jnp.float32),
        scratch_types=[
            pltpu.VMEM((kpt, _CHUNK), jnp.int32),
            pltpu.VMEM((kpt, _CHUNK), jnp.int32),
            pltpu.VMEM((2, _CHUNK), jnp.float32),
            pltpu.VMEM_SHARED((np1,), jnp.float32),
            pltpu.VMEM_SHARED((np1,), jnp.float32),
            pltpu.SemaphoreType.DMA((2,)),
            pltpu.SemaphoreType.DMA((2,)),
        ],
    )
    def scal_kernel(src_hbm, dst_hbm, v_hbm, zero_hbm, out_hbm, sbuf, dbuf,
                    vals2, v_sp, acc, gsems, ssems):
        c = lax.axis_index("c")
        s = lax.axis_index("s")
        g = s * nc + c

        @pl.when(s == 0)
        def _init():
            pltpu.sync_copy(zero_hbm, acc)

        @pl.when(s == 1)
        def _stage():
            pltpu.sync_copy(v_hbm, v_sp)

        plsc.subcore_barrier()
        pltpu.sync_copy(src_hbm.at[g], sbuf)
        pltpu.sync_copy(dst_hbm.at[g], dbuf)

        pltpu.async_copy(v_sp.at[sbuf.at[0]], vals2.at[0], gsems.at[0])

        def step(j, carry):
            slot = j % 2
            nslot = (j + 1) % 2

            @pl.when(j >= 1)
            def _drain_prev_scatter():
                pltpu.make_async_copy(v_hbm.at[pl.ds(0, _CHUNK)],
                                      vals2.at[nslot],
                                      ssems.at[nslot]).wait()

            @pl.when(j + 1 < kpt)
            def _issue_next_gather():
                pltpu.async_copy(v_sp.at[sbuf.at[j + 1]], vals2.at[nslot],
                                 gsems.at[nslot])

            pltpu.make_async_copy(v_hbm.at[pl.ds(0, _CHUNK)], vals2.at[slot],
                                  gsems.at[slot]).wait()
            pltpu.async_copy(vals2.at[slot], acc.at[dbuf.at[j]],
                             ssems.at[slot], add=True)
            return carry

        lax.fori_loop(0, kpt, step, 0)
        pltpu.make_async_copy(v_hbm.at[pl.ds(0, _CHUNK)], vals2.at[(kpt - 1) % 2],
                              ssems.at[(kpt - 1) % 2]).wait()
        plsc.subcore_barrier()

        @pl.when(s == 0)
        def _writeout():
            pltpu.sync_copy(acc, out_hbm.at[c])

    return scal_kernel


def _tc_a_body(n, deg_ref, h_ref, w1_ref, z_ref, on_ref, in_ref):
    out_deg = deg_ref[0, 0, pl.ds(0, n)] + deg_ref[1, 0, pl.ds(0, n)]
    in_deg = deg_ref[0, 1, pl.ds(0, n)] + deg_ref[1, 1, pl.ds(0, n)]
    onorm = lax.rsqrt(jnp.maximum(out_deg, 1.0))
    inorm = lax.rsqrt(jnp.maximum(in_deg, 1.0))
    on_ref[...] = onorm
    in_ref[...] = inorm
    z0 = jnp.dot(h_ref[...], w1_ref[...], preferred_element_type=jnp.float32)
    z_ref[pl.ds(0, n), :] = z0 * onorm[:, None]
    z_ref[pl.ds(n, 1), :] = jnp.zeros((1, z_ref.shape[1]), jnp.float32)


def _tc_b_body(n, agg_ref, on_ref, in_ref, b1_ref, w2_ref, v_ref):
    agg = agg_ref[0, pl.ds(0, n), :] + agg_ref[1, pl.ds(0, n), :]
    pre = agg * in_ref[...][:, None] + b1_ref[...]
    h1 = jnp.where(pre > 0, pre, jnp.exp(jnp.minimum(pre, 0.0)) - 1.0)
    w2m = jnp.mean(w2_ref[...], axis=1)
    v_ref[pl.ds(0, n)] = jnp.sum(h1 * w2m[None, :], axis=1) * on_ref[...]
    v_ref[pl.ds(n, 1)] = jnp.zeros((1,), jnp.float32)


def _tc_c_body(n, s_ref, in_ref, b2_ref, o_ref):
    o_ref[...] = ((s_ref[0, pl.ds(0, n)] + s_ref[1, pl.ds(0, n)])
                  * in_ref[...] + jnp.mean(b2_ref[...]))


def kernel(h, edge_index, W1, b1, W2, b2):
    n, _ = h.shape
    e = edge_index.shape[1]
    hid = W1.shape[1]
    np1 = n + 1
    f32 = jnp.float32

    info = plsc.get_sparse_core_info()
    nc, ns = info.num_cores, info.num_subcores
    nw = nc * ns

    per_tile = -(-e // (nw * _CHUNK)) * _CHUNK
    ep = per_tile * nw
    kpt = per_tile // _CHUNK
    if ep != e:
        pad = jnp.full((2, ep - e), n, dtype=jnp.int32)
        epad = jnp.concatenate([edge_index.astype(jnp.int32), pad], axis=1)
    else:
        epad = edge_index.astype(jnp.int32)
    src3 = epad[0].reshape(nw, kpt, _CHUNK)
    dst3 = epad[1].reshape(nw, kpt, _CHUNK)

    zeros_1 = jnp.zeros((np1,), f32)
    zeros_w = jnp.zeros((np1, hid), f32)

    deg2 = _sc_degrees(n, kpt, nc, ns)(src3, dst3, zeros_1)

    z_p, onorm, inorm = pl.pallas_call(
        functools.partial(_tc_a_body, n),
        out_shape=(
            jax.ShapeDtypeStruct((np1, hid), f32),
            jax.ShapeDtypeStruct((n,), f32),
            jax.ShapeDtypeStruct((n,), f32),
        ),
    )(deg2, h, W1)

    agg2 = _sc_edge_rows(n, hid, kpt, nc, ns)(src3, dst3, z_p, zeros_w)

    v_p = pl.pallas_call(
        functools.partial(_tc_b_body, n),
        out_shape=jax.ShapeDtypeStruct((np1,), f32),
    )(agg2, onorm, inorm, b1.reshape(1, hid), W2)

    s2 = _sc_edge_scalar(n, kpt, nc, ns)(src3, dst3, v_p, zeros_1)

    logits = pl.pallas_call(
        functools.partial(_tc_c_body, n),
        out_shape=jax.ShapeDtypeStruct((n,), f32),
    )(s2, inorm, b2.reshape(1, -1))
    return logits
